# baseline (speedup 1.0000x reference)
import functools
import jax
import jax.numpy as jnp
from jax.experimental import pallas as pl

_NB = 8
_CUT = 5.0
_AVG = 16.0


def _edge_body(ps_ref, pd_ref, hs_ref, wr_ref, msg_ref):
    d = pd_ref[...] - ps_ref[...]
    r = jnp.sqrt(jnp.sum(d * d, axis=1, keepdims=True) + 1e-12)
    rs = r + 1e-9
    n = jax.lax.broadcasted_iota(jnp.int32, (1, _NB), 1).astype(jnp.float32) + 1.0
    rb = jnp.sqrt(2.0 / _CUT) * jnp.sin(n * (jnp.pi / _CUT) * rs) / rs
    x = jnp.clip(r / _CUT, 0.0, 1.0)
    x2 = x * x
    x4 = x2 * x2
    x5 = x4 * x
    x6 = x5 * x
    x7 = x6 * x
    p = 5.0
    env = (1.0 - ((p + 1.0) * (p + 2.0) / 2.0) * x5
           + p * (p + 2.0) * x6
           - (p * (p + 1.0) / 2.0) * x7)
    rb = rb * env
    r0 = jnp.dot(rb, wr_ref[...], preferred_element_type=jnp.float32)
    msg_ref[...] = hs_ref[...] * (r0 * jax.nn.sigmoid(r0))


def _node_body(agg_ref, h_ref, wi_ref, ws_ref, wr_ref, hn_ref, er_ref):
    a = agg_ref[...] * (1.0 / _AVG)
    h = h_ref[...]
    pre = (jnp.dot(a, wi_ref[...], preferred_element_type=jnp.float32)
           + jnp.dot(h, ws_ref[...], preferred_element_type=jnp.float32))
    hn = pre * jax.nn.sigmoid(pre)
    hn_ref[...] = hn
    er_ref[...] = jnp.dot(hn, wr_ref[...], preferred_element_type=jnp.float32)


def _pick_block(total, want):
    if total % want == 0:
        return want
    return total


def kernel(positions, species, edge_index, W_embed, W_rad0, W_rad1,
           W_inter0, W_inter1, W_skip0, W_skip1, w_read0, w_read1,
           atomic_energies, scale, shift):
    n_nodes = positions.shape[0]
    E = edge_index.shape[1]
    C = W_embed.shape[1]
    src = edge_index[0]
    dst = edge_index[1]

    EB = _pick_block(E, 10000)
    NBK = _pick_block(n_nodes, 10000)

    ps = jnp.take(positions, src, axis=0)
    pd = jnp.take(positions, dst, axis=0)

    edge_call = pl.pallas_call(
        _edge_body,
        grid=(E // EB,),
        in_specs=[
            pl.BlockSpec((EB, 3), lambda i: (i, 0)),
            pl.BlockSpec((EB, 3), lambda i: (i, 0)),
            pl.BlockSpec((EB, C), lambda i: (i, 0)),
            pl.BlockSpec((_NB, C), lambda i: (0, 0)),
        ],
        out_specs=pl.BlockSpec((EB, C), lambda i: (i, 0)),
        out_shape=jax.ShapeDtypeStruct((E, C), jnp.float32),
    )

    node_call = pl.pallas_call(
        _node_body,
        grid=(n_nodes // NBK,),
        in_specs=[
            pl.BlockSpec((NBK, C), lambda i: (i, 0)),
            pl.BlockSpec((NBK, C), lambda i: (i, 0)),
            pl.BlockSpec((C, C), lambda i: (0, 0)),
            pl.BlockSpec((C, C), lambda i: (0, 0)),
            pl.BlockSpec((C, 1), lambda i: (0, 0)),
        ],
        out_specs=[
            pl.BlockSpec((NBK, C), lambda i: (i, 0)),
            pl.BlockSpec((NBK, 1), lambda i: (i, 0)),
        ],
        out_shape=[
            jax.ShapeDtypeStruct((n_nodes, C), jnp.float32),
            jax.ShapeDtypeStruct((n_nodes, 1), jnp.float32),
        ],
    )

    h = jnp.take(W_embed, species, axis=0)
    e_base = jnp.take(atomic_energies, species)
    e_readout = jnp.zeros((n_nodes,), dtype=jnp.float32)
    layers = ((W_rad0, W_inter0, W_skip0, w_read0),
              (W_rad1, W_inter1, W_skip1, w_read1))
    for (Wr, Wi, Ws, wr) in layers:
        hs = jnp.take(h, src, axis=0)
        msg = edge_call(ps, pd, hs, Wr)
        agg = jax.ops.segment_sum(msg, dst, num_segments=n_nodes)
        h, er = node_call(agg, h, Wi, Ws, wr)
        e_readout = e_readout + er[:, 0]
    return e_base + scale * e_readout + shift


# speedup vs baseline: 1.0505x; 1.0505x over previous
import functools
import jax
import jax.numpy as jnp
from jax.experimental import pallas as pl

_NB = 8
_CUT = 5.0
_AVG = 16.0


def _rb_body(ps_ref, pd_ref, rb_ref):
    d = pd_ref[...] - ps_ref[...]
    r = jnp.sqrt(jnp.sum(d * d, axis=1, keepdims=True) + 1e-12)
    rs = r + 1e-9
    n = jax.lax.broadcasted_iota(jnp.int32, (1, _NB), 1).astype(jnp.float32) + 1.0
    rb = jnp.sqrt(2.0 / _CUT) * jnp.sin(n * (jnp.pi / _CUT) * rs) / rs
    x = jnp.clip(r / _CUT, 0.0, 1.0)
    x2 = x * x
    x4 = x2 * x2
    x5 = x4 * x
    x6 = x5 * x
    x7 = x6 * x
    p = 5.0
    env = (1.0 - ((p + 1.0) * (p + 2.0) / 2.0) * x5
           + p * (p + 2.0) * x6
           - (p * (p + 1.0) / 2.0) * x7)
    rb_ref[...] = rb * env


def _msg_body(rb_ref, hs_ref, wr_ref, msg_ref):
    r0 = jnp.dot(rb_ref[...], wr_ref[...], preferred_element_type=jnp.float32)
    msg_ref[...] = hs_ref[...] * (r0 * jax.nn.sigmoid(r0))


def _node_body(agg_ref, h_ref, wi_ref, ws_ref, wr_ref, hn_ref, er_ref):
    a = agg_ref[...] * (1.0 / _AVG)
    h = h_ref[...]
    pre = (jnp.dot(a, wi_ref[...], preferred_element_type=jnp.float32)
           + jnp.dot(h, ws_ref[...], preferred_element_type=jnp.float32))
    hn = pre * jax.nn.sigmoid(pre)
    hn_ref[...] = hn
    er_ref[...] = jnp.dot(hn, wr_ref[...], preferred_element_type=jnp.float32)


def _pick_block(total, want):
    if total % want == 0:
        return want
    return total


def kernel(positions, species, edge_index, W_embed, W_rad0, W_rad1,
           W_inter0, W_inter1, W_skip0, W_skip1, w_read0, w_read1,
           atomic_energies, scale, shift):
    n_nodes = positions.shape[0]
    E = edge_index.shape[1]
    C = W_embed.shape[1]
    src = edge_index[0]
    dst = edge_index[1]

    EB = _pick_block(E, 10000)
    NBK = _pick_block(n_nodes, 10000)

    ps = jnp.take(positions, src, axis=0)
    pd = jnp.take(positions, dst, axis=0)

    rb_call = pl.pallas_call(
        _rb_body,
        grid=(E // EB,),
        in_specs=[
            pl.BlockSpec((EB, 3), lambda i: (i, 0)),
            pl.BlockSpec((EB, 3), lambda i: (i, 0)),
        ],
        out_specs=pl.BlockSpec((EB, _NB), lambda i: (i, 0)),
        out_shape=jax.ShapeDtypeStruct((E, _NB), jnp.float32),
    )
    rb = rb_call(ps, pd)

    msg_call = pl.pallas_call(
        _msg_body,
        grid=(E // EB,),
        in_specs=[
            pl.BlockSpec((EB, _NB), lambda i: (i, 0)),
            pl.BlockSpec((EB, C), lambda i: (i, 0)),
            pl.BlockSpec((_NB, C), lambda i: (0, 0)),
        ],
        out_specs=pl.BlockSpec((EB, C), lambda i: (i, 0)),
        out_shape=jax.ShapeDtypeStruct((E, C), jnp.float32),
    )

    node_call = pl.pallas_call(
        _node_body,
        grid=(n_nodes // NBK,),
        in_specs=[
            pl.BlockSpec((NBK, C), lambda i: (i, 0)),
            pl.BlockSpec((NBK, C), lambda i: (i, 0)),
            pl.BlockSpec((C, C), lambda i: (0, 0)),
            pl.BlockSpec((C, C), lambda i: (0, 0)),
            pl.BlockSpec((C, 1), lambda i: (0, 0)),
        ],
        out_specs=[
            pl.BlockSpec((NBK, C), lambda i: (i, 0)),
            pl.BlockSpec((NBK, 1), lambda i: (i, 0)),
        ],
        out_shape=[
            jax.ShapeDtypeStruct((n_nodes, C), jnp.float32),
            jax.ShapeDtypeStruct((n_nodes, 1), jnp.float32),
        ],
    )

    h = jnp.take(W_embed, species, axis=0)
    e_base = jnp.take(atomic_energies, species)
    e_readout = jnp.zeros((n_nodes,), dtype=jnp.float32)
    layers = ((W_rad0, W_inter0, W_skip0, w_read0),
              (W_rad1, W_inter1, W_skip1, w_read1))
    for (Wr, Wi, Ws, wr) in layers:
        hs = jnp.take(h, src, axis=0)
        msg = msg_call(rb, hs, Wr)
        agg = jax.ops.segment_sum(msg, dst, num_segments=n_nodes)
        h, er = node_call(agg, h, Wi, Ws, wr)
        e_readout = e_readout + er[:, 0]
    return e_base + scale * e_readout + shift


# speedup vs baseline: 1.0706x; 1.0191x over previous
import functools
import jax
import jax.numpy as jnp
from jax.experimental import pallas as pl

_NB = 8
_CUT = 5.0
_AVG = 16.0


def _edge_body(ps_ref, pd_ref, wr0_ref, wr1_ref, rad0_ref, rad1_ref):
    d = pd_ref[...] - ps_ref[...]
    r = jnp.sqrt(jnp.sum(d * d, axis=1, keepdims=True) + 1e-12)
    rs = r + 1e-9
    n = jax.lax.broadcasted_iota(jnp.int32, (1, _NB), 1).astype(jnp.float32) + 1.0
    rb = jnp.sqrt(2.0 / _CUT) * jnp.sin(n * (jnp.pi / _CUT) * rs) / rs
    x = jnp.clip(r / _CUT, 0.0, 1.0)
    x2 = x * x
    x4 = x2 * x2
    x5 = x4 * x
    x6 = x5 * x
    x7 = x6 * x
    p = 5.0
    env = (1.0 - ((p + 1.0) * (p + 2.0) / 2.0) * x5
           + p * (p + 2.0) * x6
           - (p * (p + 1.0) / 2.0) * x7)
    rb = rb * env
    r0 = jnp.dot(rb, wr0_ref[...], preferred_element_type=jnp.float32)
    r1 = jnp.dot(rb, wr1_ref[...], preferred_element_type=jnp.float32)
    rad0_ref[...] = r0 * jax.nn.sigmoid(r0)
    rad1_ref[...] = r1 * jax.nn.sigmoid(r1)


def _msg_body(hs_ref, rad_ref, msg_ref):
    msg_ref[...] = hs_ref[...] * rad_ref[...]


def _node_body(agg_ref, h_ref, wi_ref, ws_ref, wr_ref, hn_ref, er_ref):
    a = agg_ref[...] * (1.0 / _AVG)
    h = h_ref[...]
    pre = (jnp.dot(a, wi_ref[...], preferred_element_type=jnp.float32)
           + jnp.dot(h, ws_ref[...], preferred_element_type=jnp.float32))
    hn = pre * jax.nn.sigmoid(pre)
    hn_ref[...] = hn
    er_ref[...] = jnp.dot(hn, wr_ref[...], preferred_element_type=jnp.float32)


def _pick_block(total, want):
    if total % want == 0:
        return want
    return total


def kernel(positions, species, edge_index, W_embed, W_rad0, W_rad1,
           W_inter0, W_inter1, W_skip0, W_skip1, w_read0, w_read1,
           atomic_energies, scale, shift):
    n_nodes = positions.shape[0]
    E = edge_index.shape[1]
    C = W_embed.shape[1]
    src = edge_index[0]
    dst = edge_index[1]

    EB = _pick_block(E, 10000)
    NBK = _pick_block(n_nodes, 10000)

    ps = jnp.take(positions, src, axis=0)
    pd = jnp.take(positions, dst, axis=0)

    edge_call = pl.pallas_call(
        _edge_body,
        grid=(E // EB,),
        in_specs=[
            pl.BlockSpec((EB, 3), lambda i: (i, 0)),
            pl.BlockSpec((EB, 3), lambda i: (i, 0)),
            pl.BlockSpec((_NB, C), lambda i: (0, 0)),
            pl.BlockSpec((_NB, C), lambda i: (0, 0)),
        ],
        out_specs=[
            pl.BlockSpec((EB, C), lambda i: (i, 0)),
            pl.BlockSpec((EB, C), lambda i: (i, 0)),
        ],
        out_shape=[
            jax.ShapeDtypeStruct((E, C), jnp.float32),
            jax.ShapeDtypeStruct((E, C), jnp.float32),
        ],
    )
    rad0, rad1 = edge_call(ps, pd, W_rad0, W_rad1)

    msg_call = pl.pallas_call(
        _msg_body,
        grid=(E // EB,),
        in_specs=[
            pl.BlockSpec((EB, C), lambda i: (i, 0)),
            pl.BlockSpec((EB, C), lambda i: (i, 0)),
        ],
        out_specs=pl.BlockSpec((EB, C), lambda i: (i, 0)),
        out_shape=jax.ShapeDtypeStruct((E, C), jnp.float32),
    )

    node_call = pl.pallas_call(
        _node_body,
        grid=(n_nodes // NBK,),
        in_specs=[
            pl.BlockSpec((NBK, C), lambda i: (i, 0)),
            pl.BlockSpec((NBK, C), lambda i: (i, 0)),
            pl.BlockSpec((C, C), lambda i: (0, 0)),
            pl.BlockSpec((C, C), lambda i: (0, 0)),
            pl.BlockSpec((C, 1), lambda i: (0, 0)),
        ],
        out_specs=[
            pl.BlockSpec((NBK, C), lambda i: (i, 0)),
            pl.BlockSpec((NBK, 1), lambda i: (i, 0)),
        ],
        out_shape=[
            jax.ShapeDtypeStruct((n_nodes, C), jnp.float32),
            jax.ShapeDtypeStruct((n_nodes, 1), jnp.float32),
        ],
    )

    h = jnp.take(W_embed, species, axis=0)
    e_base = jnp.take(atomic_energies, species)
    e_readout = jnp.zeros((n_nodes,), dtype=jnp.float32)
    layers = ((rad0, W_inter0, W_skip0, w_read0),
              (rad1, W_inter1, W_skip1, w_read1))
    for (rad, Wi, Ws, wr) in layers:
        hs = jnp.take(h, src, axis=0)
        msg = msg_call(hs, rad)
        agg = jax.ops.segment_sum(msg, dst, num_segments=n_nodes)
        h, er = node_call(agg, h, Wi, Ws, wr)
        e_readout = e_readout + er[:, 0]
    return e_base + scale * e_readout + shift
